# baseline (speedup 1.0000x reference)
import jax
import jax.numpy as jnp
from jax.experimental import pallas as pl

_BLOCK_ROWS = 8000


def _normalize_block(x_ref, o_ref):
    x = x_ref[...]
    d = x.shape[-1]
    ones = jnp.ones((d, d), dtype=x.dtype)
    s = jax.lax.dot(x * x, ones, precision=jax.lax.Precision.HIGHEST)
    o_ref[...] = x * (1.0 / jnp.maximum(jnp.sqrt(s), 1e-12))


def kernel(prototypes):
    n, d = prototypes.shape
    grid = n // _BLOCK_ROWS
    return pl.pallas_call(
        _normalize_block,
        grid=(grid,),
        in_specs=[pl.BlockSpec((_BLOCK_ROWS, d), lambda i: (i, 0))],
        out_specs=pl.BlockSpec((_BLOCK_ROWS, d), lambda i: (i, 0)),
        out_shape=jax.ShapeDtypeStruct((n, d), prototypes.dtype),
    )(prototypes)


# speedup vs baseline: 1.1916x; 1.1916x over previous
import jax
import jax.numpy as jnp
from jax.experimental import pallas as pl

_BLOCK_ROWS = 8000


def _normalize_block(x_ref, o_ref):
    x = x_ref[...]
    s = jnp.sum(x * x, axis=-1, keepdims=True)
    o_ref[...] = x * jax.lax.rsqrt(jnp.maximum(s, 1e-24))


def kernel(prototypes):
    n, d = prototypes.shape
    grid = n // _BLOCK_ROWS
    return pl.pallas_call(
        _normalize_block,
        grid=(grid,),
        in_specs=[pl.BlockSpec((_BLOCK_ROWS, d), lambda i: (i, 0))],
        out_specs=pl.BlockSpec((_BLOCK_ROWS, d), lambda i: (i, 0)),
        out_shape=jax.ShapeDtypeStruct((n, d), prototypes.dtype),
    )(prototypes)
